# Initial kernel scaffold; baseline (speedup 1.0000x reference)
import jax
import jax.numpy as jnp
from jax.experimental import pallas as pl


def kernel(x, feature, neigh_indexs, kernels, conv_w, conv_b, bn_gamma, bn_beta):
    raise NotImplementedError("write your pallas kernel here")


# speedup vs baseline: 2.3105x; 2.3105x over previous
import functools

import jax
import jax.numpy as jnp
import numpy as np
from jax import lax
from jax.experimental import pallas as pl
from jax.experimental.pallas import tpu as pltpu
from jax.experimental.pallas import tpu_sc as plsc

B, N, IN_C, OUT_C = 8, 2048, 64, 64
K, NUM_KERNEL = 20, 16
M = B * N
TD = IN_C + NUM_KERNEL

NC, NS = 2, 16
NW = NC * NS
RPW = (M * K) // NW
SUB = 128
CH = 1024
NSUB = CH // SUB
NCH = RPW // CH
H_RPW = M // NW
H_NSUB = H_RPW // SUB

BM = 256


def _prep_body(f_ref, xp_ref, w2t_ref, w12t_ref, t_ref, h_ref):
    f = f_ref[...]
    g = jnp.dot(f, w2t_ref[...], preferred_element_type=jnp.float32)
    t_ref[...] = jnp.concatenate(
        [g, xp_ref[...], jnp.zeros((f.shape[0], TD - IN_C - 8), jnp.float32)],
        axis=1)
    h_ref[...] = jnp.dot(f, w12t_ref[...], preferred_element_type=jnp.float32)


_PREP_BLK = 2048
_prep_call = pl.pallas_call(
    _prep_body,
    grid=(M // _PREP_BLK,),
    in_specs=[
        pl.BlockSpec((_PREP_BLK, IN_C), lambda i: (i, 0)),
        pl.BlockSpec((_PREP_BLK, 8), lambda i: (i, 0)),
        pl.BlockSpec((IN_C, IN_C), lambda i: (0, 0)),
        pl.BlockSpec((IN_C, IN_C), lambda i: (0, 0)),
    ],
    out_specs=[
        pl.BlockSpec((_PREP_BLK, TD), lambda i: (i, 0)),
        pl.BlockSpec((_PREP_BLK, OUT_C), lambda i: (i, 0)),
    ],
    out_shape=[
        jax.ShapeDtypeStruct((M, TD), jnp.float32),
        jax.ShapeDtypeStruct((M, OUT_C), jnp.float32),
    ],
)


def _sc_gather_body(t_hbm, idx_hbm, h_hbm, idx0_hbm, gout, h0out,
                    idx_v, rows_v, idx0_v, h0_v, sem):
    wid = lax.axis_index("s") * NC + lax.axis_index("c")

    def chunk(gi, carry):
        pltpu.sync_copy(idx_hbm.at[pl.ds(wid * (NCH * NSUB) + gi * NSUB, NSUB)],
                        idx_v)
        copies = []
        for j in range(NSUB):
            copies.append(pltpu.async_copy(
                t_hbm.at[idx_v.at[j]],
                rows_v.at[pl.ds(j * SUB, SUB)], sem))
        for c in copies:
            c.wait()
        pltpu.sync_copy(rows_v, gout.at[pl.ds(wid * RPW + gi * CH, CH)])
        return carry

    lax.fori_loop(0, NCH, chunk, 0)

    pltpu.sync_copy(idx0_hbm.at[pl.ds(wid * H_NSUB, H_NSUB)], idx0_v)
    copies = []
    for j in range(H_NSUB):
        copies.append(pltpu.async_copy(
            h_hbm.at[idx0_v.at[j]],
            h0_v.at[pl.ds(j * SUB, SUB)], sem))
    for c in copies:
        c.wait()
    pltpu.sync_copy(h0_v, h0out.at[pl.ds(wid * H_RPW, H_RPW)])


_sc_gather = functools.partial(
    pl.kernel,
    out_type=[
        jax.ShapeDtypeStruct((M * K, TD), jnp.float32),
        jax.ShapeDtypeStruct((M, OUT_C), jnp.float32),
    ],
    mesh=plsc.VectorSubcoreMesh(core_axis_name="c", subcore_axis_name="s"),
    compiler_params=pltpu.CompilerParams(use_tc_tiling_on_sc=False),
    scratch_types=[
        pltpu.VMEM((NSUB, SUB), jnp.int32),
        pltpu.VMEM((CH, TD), jnp.float32),
        pltpu.VMEM((H_NSUB, SUB), jnp.int32),
        pltpu.VMEM((H_RPW, OUT_C), jnp.float32),
        pltpu.SemaphoreType.DMA,
    ],
)(_sc_gather_body)


def _main_body(gg_ref, h0_ref, b_ref, kp_ref, out_ref, st_ref):
    gg = gg_ref[...]
    xs = gg[:, :, IN_C:IN_C + 8]
    xrel = xs - xs[:, 0:1, :]

    def _rbf(v):
        u = lax.bitcast_convert_type(v, jnp.uint32)
        u = (u + jnp.uint32(0x7FFF) + ((u >> 16) & jnp.uint32(1)))
        u = u & jnp.uint32(0xFFFF0000)
        return lax.bitcast_convert_type(u, jnp.float32)

    xb = _rbf(xrel)
    kb = _rbf(kp_ref[...])
    pmx = (xb[:, :, 0:1] * kb[0:1, :] + xb[:, :, 1:2] * kb[1:2, :]
           + xb[:, :, 2:3] * kb[2:3, :])
    ki = lax.broadcasted_iota(jnp.int32, (1, K, NUM_KERNEL), 1)
    ji = lax.broadcasted_iota(jnp.int32, (1, K, NUM_KERNEL), 2)
    pmx = pmx + jnp.where((ki == 0) & (ji == 0), 1.0, 0.0)
    mx = jnp.max(pmx, axis=1, keepdims=True)
    e = jnp.exp(pmx - mx)
    p = e / jnp.sum(e, axis=1, keepdims=True)
    p = jnp.where(p > 0.1, p, 0.0)
    cs = jnp.sum(p, axis=1)
    pn = p / (cs[:, None, :] + 1e-6)
    cs2 = cs / (cs + 1e-6)

    g = gg[:, :, :IN_C]
    acc = pn[:, 0, :, None] * g[:, 0, None, :]
    for kk in range(1, K):
        acc = acc + pn[:, kk, :, None] * g[:, kk, None, :]
    full = h0_ref[...][:, None, :] * cs2[:, :, None] + acc
    out = jnp.max(full, axis=1) + b_ref[...]
    out_ref[...] = out

    s1 = jnp.sum(out, axis=0, keepdims=True)
    s2 = jnp.sum(out * out, axis=0, keepdims=True)
    st = jnp.concatenate([s1, s2, jnp.zeros((6, OUT_C), jnp.float32)], axis=0)

    @pl.when(pl.program_id(0) == 0)
    def _init():
        st_ref[...] = st

    @pl.when(pl.program_id(0) != 0)
    def _acc():
        st_ref[...] = st_ref[...] + st


_main_call = pl.pallas_call(
    _main_body,
    grid=(M // BM,),
    in_specs=[
        pl.BlockSpec((BM, K, TD), lambda i: (i, 0, 0)),
        pl.BlockSpec((BM, OUT_C), lambda i: (i, 0)),
        pl.BlockSpec((1, OUT_C), lambda i: (0, 0)),
        pl.BlockSpec((8, NUM_KERNEL), lambda i: (0, 0)),
    ],
    out_specs=[
        pl.BlockSpec((BM, OUT_C), lambda i: (i, 0)),
        pl.BlockSpec((8, OUT_C), lambda i: (0, 0)),
    ],
    out_shape=[
        jax.ShapeDtypeStruct((M, OUT_C), jnp.float32),
        jax.ShapeDtypeStruct((8, OUT_C), jnp.float32),
    ],
)


def _bn_body(o_ref, f_ref, st_ref, g_ref, bt_ref, out_ref):
    mean = st_ref[0:1, :] / M
    var = st_ref[1:2, :] / M - mean * mean
    rstd = lax.rsqrt(var + 1e-5)
    out_ref[...] = ((o_ref[...] - mean) * (rstd * g_ref[...])
                    + bt_ref[...] + f_ref[...])


_BN_BLK = 2048
_bn_call = pl.pallas_call(
    _bn_body,
    grid=(M // _BN_BLK,),
    in_specs=[
        pl.BlockSpec((_BN_BLK, OUT_C), lambda i: (i, 0)),
        pl.BlockSpec((_BN_BLK, OUT_C), lambda i: (i, 0)),
        pl.BlockSpec((8, OUT_C), lambda i: (0, 0)),
        pl.BlockSpec((1, OUT_C), lambda i: (0, 0)),
        pl.BlockSpec((1, OUT_C), lambda i: (0, 0)),
    ],
    out_specs=pl.BlockSpec((_BN_BLK, OUT_C), lambda i: (i, 0)),
    out_shape=jax.ShapeDtypeStruct((M, OUT_C), jnp.float32),
)

_PERM = (np.arange(2 * IN_C) % 32) * 4 + (np.arange(2 * IN_C) // 32)


def kernel(x, feature, neigh_indexs, kernels, conv_w, conv_b,
           bn_gamma, bn_beta):
    f = jnp.transpose(feature, (0, 2, 1)).reshape(M, IN_C)
    xp = jnp.transpose(x, (0, 2, 1)).reshape(M, 3)
    xp8 = jnp.pad(xp, ((0, 0), (0, 5)))
    kp = jnp.pad(kernels, ((0, 5), (0, 0)))

    wt = conv_w[:, _PERM]
    w1, w2 = wt[:, :IN_C], wt[:, IN_C:]
    w2t = jnp.transpose(w2)
    w12t = jnp.transpose(w1 - w2)

    idxb = (neigh_indexs.astype(jnp.int32)
            + (jnp.arange(B, dtype=jnp.int32) * N)[:, None, None])
    idx2d = idxb.reshape((M * K) // SUB, SUB)
    idx02d = idxb[:, :, 0].reshape(M // SUB, SUB)

    tbl, h = _prep_call(f, xp8, w2t, w12t)
    gg, h0 = _sc_gather(tbl, idx2d, h, idx02d)
    outp, stats = _main_call(gg.reshape(M, K, TD), h0,
                             conv_b.reshape(1, OUT_C), kp)
    fin = _bn_call(outp, f, stats, bn_gamma.reshape(1, OUT_C),
                   bn_beta.reshape(1, OUT_C))
    return jnp.transpose(fin.reshape(B, N, OUT_C), (0, 2, 1))


# speedup vs baseline: 3.2397x; 1.4022x over previous
import functools

import jax
import jax.numpy as jnp
import numpy as np
from jax import lax
from jax.experimental import pallas as pl
from jax.experimental.pallas import tpu as pltpu
from jax.experimental.pallas import tpu_sc as plsc

B, N, IN_C, OUT_C = 8, 2048, 64, 64
K, NUM_KERNEL = 20, 16
M = B * N
TD = IN_C + NUM_KERNEL

NC, NS = 2, 16
NW = NC * NS
RPW = (M * K) // NW
SUB = 128
CH = 1024
NSUB = CH // SUB
NCH = RPW // CH
H_RPW = M // NW
H_NSUB = H_RPW // SUB

BM = 256


def _prep_body(f_ref, xp_ref, w2t_ref, w12t_ref, t_ref, h_ref):
    f = f_ref[...]
    g = jnp.dot(f, w2t_ref[...], preferred_element_type=jnp.float32)
    t_ref[...] = jnp.concatenate(
        [g, xp_ref[...], jnp.zeros((f.shape[0], TD - IN_C - 8), jnp.float32)],
        axis=1)
    h_ref[...] = jnp.dot(f, w12t_ref[...], preferred_element_type=jnp.float32)


_PREP_BLK = 2048
_prep_call = pl.pallas_call(
    _prep_body,
    grid=(M // _PREP_BLK,),
    in_specs=[
        pl.BlockSpec((_PREP_BLK, IN_C), lambda i: (i, 0)),
        pl.BlockSpec((_PREP_BLK, 8), lambda i: (i, 0)),
        pl.BlockSpec((IN_C, IN_C), lambda i: (0, 0)),
        pl.BlockSpec((IN_C, IN_C), lambda i: (0, 0)),
    ],
    out_specs=[
        pl.BlockSpec((_PREP_BLK, TD), lambda i: (i, 0)),
        pl.BlockSpec((_PREP_BLK, OUT_C), lambda i: (i, 0)),
    ],
    out_shape=[
        jax.ShapeDtypeStruct((M, TD), jnp.float32),
        jax.ShapeDtypeStruct((M, OUT_C), jnp.float32),
    ],
)


def _sc_gather_body(t_hbm, idx_hbm, h_hbm, idx0_hbm, gout, h0out,
                    idx_v, rows_v, idx0_v, h0_v, sem):
    wid = lax.axis_index("s") * NC + lax.axis_index("c")

    def chunk(gi, carry):
        pltpu.sync_copy(idx_hbm.at[pl.ds(wid * (NCH * NSUB) + gi * NSUB, NSUB)],
                        idx_v)
        copies = []
        for j in range(NSUB):
            copies.append(pltpu.async_copy(
                t_hbm.at[idx_v.at[j]],
                rows_v.at[pl.ds(j * SUB, SUB)], sem))
        for c in copies:
            c.wait()
        pltpu.sync_copy(rows_v, gout.at[pl.ds(wid * RPW + gi * CH, CH)])
        return carry

    lax.fori_loop(0, NCH, chunk, 0)

    pltpu.sync_copy(idx0_hbm.at[pl.ds(wid * H_NSUB, H_NSUB)], idx0_v)
    copies = []
    for j in range(H_NSUB):
        copies.append(pltpu.async_copy(
            h_hbm.at[idx0_v.at[j]],
            h0_v.at[pl.ds(j * SUB, SUB)], sem))
    for c in copies:
        c.wait()
    pltpu.sync_copy(h0_v, h0out.at[pl.ds(wid * H_RPW, H_RPW)])


_sc_gather = functools.partial(
    pl.kernel,
    out_type=[
        jax.ShapeDtypeStruct((M * K, TD), jnp.float32),
        jax.ShapeDtypeStruct((M, OUT_C), jnp.float32),
    ],
    mesh=plsc.VectorSubcoreMesh(core_axis_name="c", subcore_axis_name="s"),
    compiler_params=pltpu.CompilerParams(use_tc_tiling_on_sc=False),
    scratch_types=[
        pltpu.VMEM((NSUB, SUB), jnp.int32),
        pltpu.VMEM((CH, TD), jnp.float32),
        pltpu.VMEM((H_NSUB, SUB), jnp.int32),
        pltpu.VMEM((H_RPW, OUT_C), jnp.float32),
        pltpu.SemaphoreType.DMA,
    ],
)(_sc_gather_body)


def _main_body(gg_ref, h0_ref, b_ref, kp_ref, out_ref, st_ref):
    gg = gg_ref[...]
    xs = gg[:, :, IN_C:IN_C + 8]
    xrel = xs - xs[:, 0:1, :]

    def _rbf(v):
        u = lax.bitcast_convert_type(v, jnp.uint32)
        u = (u + jnp.uint32(0x7FFF) + ((u >> 16) & jnp.uint32(1)))
        u = u & jnp.uint32(0xFFFF0000)
        return lax.bitcast_convert_type(u, jnp.float32)

    xb = _rbf(xrel)
    kb = _rbf(kp_ref[...])
    pmx = (xb[:, :, 0:1] * kb[0:1, :] + xb[:, :, 1:2] * kb[1:2, :]
           + xb[:, :, 2:3] * kb[2:3, :])
    ki = lax.broadcasted_iota(jnp.int32, (1, K, NUM_KERNEL), 1)
    ji = lax.broadcasted_iota(jnp.int32, (1, K, NUM_KERNEL), 2)
    pmx = pmx + jnp.where((ki == 0) & (ji == 0), 1.0, 0.0)
    mx = jnp.max(pmx, axis=1, keepdims=True)
    e = jnp.exp(pmx - mx)
    p = e / jnp.sum(e, axis=1, keepdims=True)
    p = jnp.where(p > 0.1, p, 0.0)
    cs = jnp.sum(p, axis=1)
    pn = p / (cs[:, None, :] + 1e-6)
    cs2 = cs / (cs + 1e-6)

    g = gg[:, :, :IN_C]
    JC = NUM_KERNEL * IN_C
    ji = lax.broadcasted_iota(jnp.int32, (NUM_KERNEL, JC), 0)
    li = lax.broadcasted_iota(jnp.int32, (NUM_KERNEL, JC), 1)
    rep_j = jnp.where(li // IN_C == ji, 1.0, 0.0)
    ci = lax.broadcasted_iota(jnp.int32, (IN_C, JC), 0)
    li2 = lax.broadcasted_iota(jnp.int32, (IN_C, JC), 1)
    rep_c = jnp.where(li2 % IN_C == ci, 1.0, 0.0)

    acc = None
    for kk in range(K):
        pr = jnp.dot(pn[:, kk, :], rep_j, preferred_element_type=jnp.float32)
        gr = jnp.dot(g[:, kk, :], rep_c, preferred_element_type=jnp.float32)
        acc = pr * gr if acc is None else acc + pr * gr
    csr = jnp.dot(cs2, rep_j, preferred_element_type=jnp.float32)
    h0r = jnp.dot(h0_ref[...], rep_c, preferred_element_type=jnp.float32)
    full = h0r * csr + acc
    out = full[:, :IN_C]
    for jj in range(1, NUM_KERNEL):
        out = jnp.maximum(out, full[:, jj * IN_C:(jj + 1) * IN_C])
    out = out + b_ref[...]
    out_ref[...] = out

    s1 = jnp.sum(out, axis=0, keepdims=True)
    s2 = jnp.sum(out * out, axis=0, keepdims=True)
    st = jnp.concatenate([s1, s2, jnp.zeros((6, OUT_C), jnp.float32)], axis=0)

    @pl.when(pl.program_id(0) == 0)
    def _init():
        st_ref[...] = st

    @pl.when(pl.program_id(0) != 0)
    def _acc():
        st_ref[...] = st_ref[...] + st


_main_call = pl.pallas_call(
    _main_body,
    grid=(M // BM,),
    in_specs=[
        pl.BlockSpec((BM, K, TD), lambda i: (i, 0, 0)),
        pl.BlockSpec((BM, OUT_C), lambda i: (i, 0)),
        pl.BlockSpec((1, OUT_C), lambda i: (0, 0)),
        pl.BlockSpec((8, NUM_KERNEL), lambda i: (0, 0)),
    ],
    out_specs=[
        pl.BlockSpec((BM, OUT_C), lambda i: (i, 0)),
        pl.BlockSpec((8, OUT_C), lambda i: (0, 0)),
    ],
    out_shape=[
        jax.ShapeDtypeStruct((M, OUT_C), jnp.float32),
        jax.ShapeDtypeStruct((8, OUT_C), jnp.float32),
    ],
)


def _bn_body(o_ref, f_ref, st_ref, g_ref, bt_ref, out_ref):
    mean = st_ref[0:1, :] / M
    var = st_ref[1:2, :] / M - mean * mean
    rstd = lax.rsqrt(var + 1e-5)
    out_ref[...] = ((o_ref[...] - mean) * (rstd * g_ref[...])
                    + bt_ref[...] + f_ref[...])


_BN_BLK = 2048
_bn_call = pl.pallas_call(
    _bn_body,
    grid=(M // _BN_BLK,),
    in_specs=[
        pl.BlockSpec((_BN_BLK, OUT_C), lambda i: (i, 0)),
        pl.BlockSpec((_BN_BLK, OUT_C), lambda i: (i, 0)),
        pl.BlockSpec((8, OUT_C), lambda i: (0, 0)),
        pl.BlockSpec((1, OUT_C), lambda i: (0, 0)),
        pl.BlockSpec((1, OUT_C), lambda i: (0, 0)),
    ],
    out_specs=pl.BlockSpec((_BN_BLK, OUT_C), lambda i: (i, 0)),
    out_shape=jax.ShapeDtypeStruct((M, OUT_C), jnp.float32),
)

_PERM = (np.arange(2 * IN_C) % 32) * 4 + (np.arange(2 * IN_C) // 32)


def kernel(x, feature, neigh_indexs, kernels, conv_w, conv_b,
           bn_gamma, bn_beta):
    f = jnp.transpose(feature, (0, 2, 1)).reshape(M, IN_C)
    xp = jnp.transpose(x, (0, 2, 1)).reshape(M, 3)
    xp8 = jnp.pad(xp, ((0, 0), (0, 5)))
    kp = jnp.pad(kernels, ((0, 5), (0, 0)))

    wt = conv_w[:, _PERM]
    w1, w2 = wt[:, :IN_C], wt[:, IN_C:]
    w2t = jnp.transpose(w2)
    w12t = jnp.transpose(w1 - w2)

    idxb = (neigh_indexs.astype(jnp.int32)
            + (jnp.arange(B, dtype=jnp.int32) * N)[:, None, None])
    idx2d = idxb.reshape((M * K) // SUB, SUB)
    idx02d = idxb[:, :, 0].reshape(M // SUB, SUB)

    tbl, h = _prep_call(f, xp8, w2t, w12t)
    gg, h0 = _sc_gather(tbl, idx2d, h, idx02d)
    outp, stats = _main_call(gg.reshape(M, K, TD), h0,
                             conv_b.reshape(1, OUT_C), kp)
    fin = _bn_call(outp, f, stats, bn_gamma.reshape(1, OUT_C),
                   bn_beta.reshape(1, OUT_C))
    return jnp.transpose(fin.reshape(B, N, OUT_C), (0, 2, 1))
